# baseline (speedup 1.0000x reference)
import functools

import jax
import jax.numpy as jnp
from jax import lax
from jax.experimental import pallas as pl
from jax.experimental.pallas import tpu as pltpu
from jax.experimental.pallas import tpu_sc as plsc

B = 16384
L = 200
VOCAB = 1000
D = 64

NC = 2
NS = 16
NW = NC * NS
LANES = 16

ROWS_PER_W = B // NW
GROUPS = ROWS_PER_W // LANES
JCHUNK = 40
NJC = L // JCHUNK

DSLICE = D // 8
VGF = VOCAB // LANES
VTAIL = VOCAB - LANES


def _vocab_groups():
    return [g * LANES for g in range(VGF)] + [VTAIL]


def _pool_body(xt_hbm, embt_hbm, w_hbm, b_hbm, out_hbm,
               xa_v, xb_v, e_v, p_v, s_v, o_v, w_v, b_v, r_v, shared,
               sem_a, sem_b):
    cid = lax.axis_index("c")
    sid = lax.axis_index("s")
    wid = sid * NC + cid
    base = wid * ROWS_PER_W

    bufs = (xa_v, xb_v)
    sems = (sem_a, sem_b)

    def start(c):
        return pltpu.async_copy(
            xt_hbm.at[pl.ds(c * JCHUNK, JCHUNK), pl.ds(base, ROWS_PER_W)],
            bufs[c % 2], sems[c % 2])

    handles = [start(0), start(1)]

    pltpu.sync_copy(w_hbm, w_v)
    pltpu.sync_copy(b_hbm, b_v)

    zero = jnp.zeros((LANES,), jnp.int32)

    @pl.when(sid < 8)
    def _table():
        pltpu.sync_copy(embt_hbm.at[pl.ds(sid * DSLICE, DSLICE), :], e_v)
        wspl = [plsc.load_gather(w_v, [zero + (sid * DSLICE + d)])
                for d in range(DSLICE)]
        for v0 in _vocab_groups():
            acc = jnp.zeros((LANES,), jnp.float32)
            for d in range(DSLICE):
                acc = acc + e_v[d, pl.ds(v0, LANES)] * wspl[d]
            p_v[pl.ds(v0, LANES)] = acc
        pltpu.sync_copy(p_v, shared.at[pl.ds(sid * VOCAB, VOCAB)])

    plsc.subcore_barrier()

    @pl.when(sid == 0)
    def _reduce():
        pltpu.sync_copy(shared.at[pl.ds(0, 8 * VOCAB)], r_v)
        for v0 in _vocab_groups():
            acc = r_v[pl.ds(v0, LANES)]
            for d in range(1, 8):
                acc = acc + r_v[pl.ds(d * VOCAB + v0, LANES)]
            p_v[pl.ds(v0, LANES)] = acc * (1.0 / L)
        pltpu.sync_copy(p_v, shared.at[pl.ds(8 * VOCAB, VOCAB)])

    plsc.subcore_barrier()
    pltpu.sync_copy(shared.at[pl.ds(8 * VOCAB, VOCAB)], s_v)
    b_spl = plsc.load_gather(b_v, [zero])

    for c in range(NJC):
        handles[c].wait()
        x_v = bufs[c % 2]
        first, last = c == 0, c == NJC - 1

        def group_body(g, carry):
            i0 = g * LANES

            def j_body(j, acc):
                xi = x_v[j, pl.ds(i0, LANES)]
                return acc + plsc.load_gather(s_v, [xi])

            acc = lax.fori_loop(0, JCHUNK, j_body,
                                jnp.zeros((LANES,), jnp.float32), unroll=8)
            if not first:
                acc = acc + o_v[pl.ds(i0, LANES)]
            if last:
                acc = 1.0 / (1.0 + jnp.exp(-(acc + b_spl)))
            o_v[pl.ds(i0, LANES)] = acc
            return carry

        lax.fori_loop(0, GROUPS, group_body, 0)
        if c + 2 < NJC:
            handles.append(start(c + 2))

    pltpu.sync_copy(o_v, out_hbm.at[pl.ds(base, ROWS_PER_W)])


def kernel(x, emb, W, b):
    mesh = plsc.VectorSubcoreMesh(core_axis_name="c", subcore_axis_name="s")
    pool = functools.partial(
        pl.kernel,
        out_type=jax.ShapeDtypeStruct((B,), jnp.float32),
        mesh=mesh,
        scratch_types=[
            pltpu.VMEM((JCHUNK, ROWS_PER_W), jnp.int32),
            pltpu.VMEM((JCHUNK, ROWS_PER_W), jnp.int32),
            pltpu.VMEM((DSLICE, VOCAB), jnp.float32),
            pltpu.VMEM((VOCAB,), jnp.float32),
            pltpu.VMEM((VOCAB,), jnp.float32),
            pltpu.VMEM((ROWS_PER_W,), jnp.float32),
            pltpu.VMEM((D,), jnp.float32),
            pltpu.VMEM((1,), jnp.float32),
            pltpu.VMEM((8 * VOCAB,), jnp.float32),
            pltpu.VMEM_SHARED((9 * VOCAB,), jnp.float32),
            pltpu.SemaphoreType.DMA,
            pltpu.SemaphoreType.DMA,
        ],
        compiler_params=pltpu.CompilerParams(needs_layout_passes=False),
    )(_pool_body)
    out = pool(x.T.astype(jnp.int32), emb.T.astype(jnp.float32),
               W.reshape(D).astype(jnp.float32), b.astype(jnp.float32))
    return out.reshape(B, 1)


# speedup vs baseline: 1.1775x; 1.1775x over previous
import functools

import jax
import jax.numpy as jnp
from jax import lax
from jax.experimental import pallas as pl
from jax.experimental.pallas import tpu as pltpu
from jax.experimental.pallas import tpu_sc as plsc

B = 16384
L = 200
VOCAB = 1000
D = 64

NC = 2
NS = 16
NW = NC * NS
LANES = 16

ROWS_PER_W = B // NW
GROUPS = ROWS_PER_W // LANES
JCHUNK = 40
NJC = L // JCHUNK


def _table_kernel(emb_t_ref, w_ref, b_ref, s_ref):
    prod = emb_t_ref[...] * w_ref[...][:, None]
    s = jnp.sum(prod, axis=0)
    s_ref[...] = (s + b_ref[0]) * (1.0 / L)


def _pool_body(xt_hbm, s_hbm, out_hbm, xa_v, xb_v, s_v, o_v, sem_a, sem_b):
    cid = lax.axis_index("c")
    sid = lax.axis_index("s")
    wid = sid * NC + cid
    base = wid * ROWS_PER_W

    bufs = (xa_v, xb_v)
    sems = (sem_a, sem_b)

    def start(c):
        return pltpu.async_copy(
            xt_hbm.at[pl.ds(c * JCHUNK, JCHUNK), pl.ds(base, ROWS_PER_W)],
            bufs[c % 2], sems[c % 2])

    handles = [start(0), start(1)]
    pltpu.sync_copy(s_hbm, s_v)

    for c in range(NJC):
        handles[c].wait()
        x_v = bufs[c % 2]
        first, last = c == 0, c == NJC - 1

        def group_body(g, carry):
            i0 = g * LANES

            def j_body(j, acc):
                xi = x_v[j, pl.ds(i0, LANES)]
                return acc + plsc.load_gather(s_v, [xi])

            acc = lax.fori_loop(0, JCHUNK, j_body,
                                jnp.zeros((LANES,), jnp.float32), unroll=8)
            if not first:
                acc = acc + o_v[pl.ds(i0, LANES)]
            if last:
                acc = 1.0 / (1.0 + jnp.exp(-acc))
            o_v[pl.ds(i0, LANES)] = acc
            return carry

        lax.fori_loop(0, GROUPS, group_body, 0)
        if c + 2 < NJC:
            handles.append(start(c + 2))

    pltpu.sync_copy(o_v, out_hbm.at[pl.ds(base, ROWS_PER_W)])


def kernel(x, emb, W, b):
    w = W.reshape(D).astype(jnp.float32)
    s_flat = pl.pallas_call(
        _table_kernel,
        out_shape=jax.ShapeDtypeStruct((VOCAB,), jnp.float32),
    )(emb.T, w, b.astype(jnp.float32))

    mesh = plsc.VectorSubcoreMesh(core_axis_name="c", subcore_axis_name="s")
    pool = functools.partial(
        pl.kernel,
        out_type=jax.ShapeDtypeStruct((B,), jnp.float32),
        mesh=mesh,
        scratch_types=[
            pltpu.VMEM((JCHUNK, ROWS_PER_W), jnp.int32),
            pltpu.VMEM((JCHUNK, ROWS_PER_W), jnp.int32),
            pltpu.VMEM((VOCAB,), jnp.float32),
            pltpu.VMEM((ROWS_PER_W,), jnp.float32),
            pltpu.SemaphoreType.DMA,
            pltpu.SemaphoreType.DMA,
        ],
        compiler_params=pltpu.CompilerParams(needs_layout_passes=False),
    )(_pool_body)
    out = pool(x.T.astype(jnp.int32), s_flat)
    return out.reshape(B, 1)
